# baseline (speedup 1.0000x reference)
import jax, jax.numpy as jnp
from jax import lax
from jax.experimental import pallas as pl
from jax.experimental.pallas import tpu as pltpu, tpu_sc as plsc

NC, NS = 2, 16
NW = NC * NS
DEPTH = 4
TC_BLOCK = 1024
SC_FRACTION_NUM, SC_FRACTION_DEN = 55, 100


def _sc_body(ids_hbm, table_hbm, out_hbm, idx_all, tab_v, sem):
    wid = lax.axis_index("s") * NC + lax.axis_index("c")
    n = ids_hbm.shape[0]
    pw = n // NW
    base = wid * pw
    ngroups = pw // 16

    pltpu.sync_copy(ids_hbm.at[pl.ds(base, pw)], idx_all)
    pltpu.sync_copy(table_hbm, tab_v)

    def issue_group(g):
        ids_v = idx_all[pl.ds(g * 16, 16)]
        goff = base + g * 16
        for l in range(16):
            tid = ids_v[l]
            pltpu.async_copy(tab_v.at[pl.ds(tid, 1)],
                             out_hbm.at[pl.ds(goff + l, 1)], sem)

    def drain_group():
        pltpu.make_async_copy(tab_v.at[pl.ds(0, 16)],
                              out_hbm.at[pl.ds(0, 16)], sem).wait()

    @pl.loop(0, DEPTH)
    def _prime(g):
        issue_group(g)

    @pl.loop(DEPTH, ngroups)
    def _steady(g):
        drain_group()
        issue_group(g)

    @pl.loop(0, DEPTH)
    def _tail(g):
        drain_group()


def _tc_body(ids_ref, table_ref, out_ref):
    ids = ids_ref[0, 0, :]
    nv = table_ref.shape[0]
    onehot = (ids[:, None] == jax.lax.broadcasted_iota(jnp.int32, (TC_BLOCK, nv), 1))
    out_ref[...] = jnp.dot(onehot.astype(jnp.float32), table_ref[...],
                           preferred_element_type=jnp.float32)


def kernel(token_ids, table):
    b, s = token_ids.shape
    v, d = table.shape
    ids = token_ids.reshape(-1).astype(jnp.int32)
    n = ids.shape[0]

    import math
    align = NW * 16 * TC_BLOCK // math.gcd(NW * 16, TC_BLOCK)
    n_sc = (n * SC_FRACTION_NUM // SC_FRACTION_DEN) // align * align
    n_tc = n - n_sc

    mesh = plsc.VectorSubcoreMesh(core_axis_name="c", subcore_axis_name="s",
                                  num_cores=NC, num_subcores=NS)
    out_sc = pl.kernel(
        _sc_body, out_type=jax.ShapeDtypeStruct((n_sc, d), jnp.float32),
        mesh=mesh,
        compiler_params=pltpu.CompilerParams(needs_layout_passes=False),
        scratch_types=[
            pltpu.VMEM((n_sc // NW,), jnp.int32),
            pltpu.VMEM((v, d), jnp.float32),
            pltpu.SemaphoreType.DMA,
        ],
    )(ids[:n_sc], table)

    nb = n_tc // TC_BLOCK
    out_tc = pl.pallas_call(
        _tc_body,
        grid=(nb,),
        in_specs=[
            pl.BlockSpec((1, 1, TC_BLOCK), lambda i: (i, 0, 0)),
            pl.BlockSpec((v, d), lambda i: (0, 0)),
        ],
        out_specs=pl.BlockSpec((TC_BLOCK, d), lambda i: (i, 0)),
        out_shape=jax.ShapeDtypeStruct((n_tc, d), jnp.float32),
    )(ids[n_sc:].reshape(nb, 1, TC_BLOCK), table)

    return jnp.concatenate([out_sc, out_tc], axis=0).reshape(b, s, d)


# speedup vs baseline: 2.6323x; 2.6323x over previous
import jax, jax.numpy as jnp
from jax import lax
from jax.experimental import pallas as pl
from jax.experimental.pallas import tpu as pltpu, tpu_sc as plsc

NC, NS = 2, 16
NW = NC * NS
DEPTH = 4


def body(ids_hbm, table_hbm, out_hbm, idx_all, tab_v, sem):
    wid = lax.axis_index("s") * NC + lax.axis_index("c")
    n = ids_hbm.shape[0]
    pw = n // NW
    base = wid * pw
    ngroups = pw // 16

    pltpu.sync_copy(ids_hbm.at[pl.ds(base, pw)], idx_all)
    pltpu.sync_copy(table_hbm, tab_v)

    def issue_group(g):
        ids_v = idx_all[pl.ds(g * 16, 16)]
        goff = base + g * 16
        for l in range(16):
            tid = ids_v[l]
            pltpu.async_copy(tab_v.at[pl.ds(tid, 1)],
                             out_hbm.at[pl.ds(goff + l, 1)], sem)

    def drain_group():
        pltpu.make_async_copy(tab_v.at[pl.ds(0, 16)],
                              out_hbm.at[pl.ds(0, 16)], sem).wait()

    @pl.loop(0, DEPTH)
    def _prime(g):
        issue_group(g)

    @pl.loop(DEPTH, ngroups)
    def _steady(g):
        drain_group()
        issue_group(g)

    @pl.loop(0, DEPTH)
    def _tail(g):
        drain_group()


def kernel(token_ids, table):
    b, s = token_ids.shape
    v, d = table.shape
    ids = token_ids.reshape(-1).astype(jnp.int32)
    n = ids.shape[0]
    mesh = plsc.VectorSubcoreMesh(core_axis_name="c", subcore_axis_name="s",
                                  num_cores=NC, num_subcores=NS)
    out = pl.kernel(
        body, out_type=jax.ShapeDtypeStruct((n, d), jnp.float32), mesh=mesh,
        compiler_params=pltpu.CompilerParams(needs_layout_passes=False),
        scratch_types=[
            pltpu.VMEM((n // NW,), jnp.int32),
            pltpu.VMEM((v, d), jnp.float32),
            pltpu.SemaphoreType.DMA,
        ],
    )(ids, table)
    return out.reshape(b, s, d)
